# baseline (speedup 1.0000x reference)
import functools

import jax
import jax.numpy as jnp
from jax.experimental import pallas as pl


@functools.partial(jax.jit, static_argnums=(0, 1, 2))
def _pos_embed_tc(B, H, W, row_embed, col_embed):
    D = row_embed.shape[1]

    HB = 8

    def body(row_ref, col_ref, o_ref):
        q = pl.program_id(1)
        col = col_ref[:W, :]
        row = row_ref[pl.ds(q * HB, HB), :]
        o_ref[0, :, :, :D] = jnp.broadcast_to(col[None, :, :], (HB, W, D))
        o_ref[0, :, :, D:] = jnp.broadcast_to(row[:, None, :], (HB, W, D))

    out = pl.pallas_call(
        body,
        grid=(B, H // HB),
        in_specs=[
            pl.BlockSpec(row_embed.shape, lambda b, q: (0, 0)),
            pl.BlockSpec(col_embed.shape, lambda b, q: (0, 0)),
        ],
        out_specs=pl.BlockSpec((1, HB, W, 2 * D), lambda b, q: (b, q, 0, 0)),
        out_shape=jax.ShapeDtypeStruct((B, H, W, 2 * D), jnp.float32),
    )(row_embed, col_embed)
    return jnp.transpose(out, (0, 3, 1, 2))


def kernel(x, row_embed, col_embed):
    B = x.shape[0]
    H, W = x.shape[-2], x.shape[-1]
    return _pos_embed_tc(B, H, W, row_embed, col_embed)


# speedup vs baseline: 1.6409x; 1.6409x over previous
import functools

import jax
import jax.numpy as jnp
from jax.experimental import pallas as pl


@functools.partial(jax.jit, static_argnums=(0, 1, 2))
def _pos_embed_tc(B, H, W, row_embed, col_embed):
    D = row_embed.shape[1]

    def body(row_ref, col_ref, o_ref):
        col = col_ref[:W, :]
        row = row_ref[:H, :]
        o_ref[:, :, :, :D] = jnp.broadcast_to(col[None, None, :, :],
                                              (B, H, W, D))
        o_ref[:, :, :, D:] = jnp.broadcast_to(row[None, :, None, :],
                                              (B, H, W, D))

    out = pl.pallas_call(
        body,
        in_specs=[
            pl.BlockSpec(row_embed.shape, lambda: (0, 0)),
            pl.BlockSpec(col_embed.shape, lambda: (0, 0)),
        ],
        out_specs=pl.BlockSpec((B, H, W, 2 * D), lambda: (0, 0, 0, 0)),
        out_shape=jax.ShapeDtypeStruct((B, H, W, 2 * D), jnp.float32),
    )(row_embed, col_embed)
    return jnp.transpose(out, (0, 3, 1, 2))


def kernel(x, row_embed, col_embed):
    B = x.shape[0]
    H, W = x.shape[-2], x.shape[-1]
    return _pos_embed_tc(B, H, W, row_embed, col_embed)


# speedup vs baseline: 1.8560x; 1.1310x over previous
import functools

import jax
import jax.numpy as jnp
from jax.experimental import pallas as pl
from jax.experimental.pallas import tpu as pltpu


@functools.partial(jax.jit, static_argnums=(0, 1, 2))
def _pos_embed_tc(B, H, W, row_embed, col_embed):
    D = row_embed.shape[1]

    def body(row_ref, col_ref, o_hbm, scratch, sems):
        col = col_ref[:W, :]
        row = row_ref[:H, :]
        scratch[:, :, :D] = jnp.broadcast_to(col[None, :, :], (H, W, D))
        scratch[:, :, D:] = jnp.broadcast_to(row[:, None, :], (H, W, D))
        copies = [
            pltpu.make_async_copy(scratch, o_hbm.at[b], sems.at[b])
            for b in range(B)
        ]
        for c in copies:
            c.start()
        for c in copies:
            c.wait()

    out = pl.pallas_call(
        body,
        in_specs=[
            pl.BlockSpec(row_embed.shape, lambda: (0, 0)),
            pl.BlockSpec(col_embed.shape, lambda: (0, 0)),
        ],
        out_specs=pl.BlockSpec(memory_space=pl.ANY),
        out_shape=jax.ShapeDtypeStruct((B, H, W, 2 * D), jnp.float32),
        scratch_shapes=[
            pltpu.VMEM((H, W, 2 * D), jnp.float32),
            pltpu.SemaphoreType.DMA((B,)),
        ],
    )(row_embed, col_embed)
    return jnp.transpose(out, (0, 3, 1, 2))


def kernel(x, row_embed, col_embed):
    B = x.shape[0]
    H, W = x.shape[-2], x.shape[-1]
    return _pos_embed_tc(B, H, W, row_embed, col_embed)


# speedup vs baseline: 1.8643x; 1.0045x over previous
import functools

import jax
import jax.numpy as jnp
from jax.experimental import pallas as pl
from jax.experimental.pallas import tpu as pltpu


@functools.partial(jax.jit, static_argnums=(0, 1, 2))
def _pos_embed_tc(B, H, W, row_embed, col_embed):
    D = row_embed.shape[1]

    HH = H // 2

    def body(row_ref, col_ref, o_hbm, scratch, sems):
        col = col_ref[:W, :]
        copies = []
        for h in range(2):
            sl = pl.ds(h * HH, HH)
            row = row_ref[sl, :]
            scratch[sl, :, :D] = jnp.broadcast_to(col[None, :, :],
                                                  (HH, W, D))
            scratch[sl, :, D:] = jnp.broadcast_to(row[:, None, :],
                                                  (HH, W, D))
            for b in range(B):
                c = pltpu.make_async_copy(
                    scratch.at[sl], o_hbm.at[b, sl], sems.at[h, b])
                c.start()
                copies.append(c)
        for c in copies:
            c.wait()

    out = pl.pallas_call(
        body,
        in_specs=[
            pl.BlockSpec(row_embed.shape, lambda: (0, 0)),
            pl.BlockSpec(col_embed.shape, lambda: (0, 0)),
        ],
        out_specs=pl.BlockSpec(memory_space=pl.ANY),
        out_shape=jax.ShapeDtypeStruct((B, H, W, 2 * D), jnp.float32),
        scratch_shapes=[
            pltpu.VMEM((H, W, 2 * D), jnp.float32),
            pltpu.SemaphoreType.DMA((2, B)),
        ],
    )(row_embed, col_embed)
    return jnp.transpose(out, (0, 3, 1, 2))


def kernel(x, row_embed, col_embed):
    B = x.shape[0]
    H, W = x.shape[-2], x.shape[-1]
    return _pos_embed_tc(B, H, W, row_embed, col_embed)


# speedup vs baseline: 1.8757x; 1.0061x over previous
import functools

import jax
import jax.numpy as jnp
from jax.experimental import pallas as pl
from jax.experimental.pallas import tpu as pltpu


@functools.partial(jax.jit, static_argnums=(0, 1, 2))
def _pos_embed_tc(B, H, W, row_embed, col_embed):
    D = row_embed.shape[1]

    NS = 4
    HH = H // NS

    def body(row_ref, col_ref, o_hbm, scratch, sems):
        col = col_ref[:W, :]
        copies = []
        for h in range(NS):
            sl = pl.ds(h * HH, HH)
            row = row_ref[sl, :]
            scratch[sl, :, :D] = jnp.broadcast_to(col[None, :, :],
                                                  (HH, W, D))
            scratch[sl, :, D:] = jnp.broadcast_to(row[:, None, :],
                                                  (HH, W, D))
            for b in range(B):
                c = pltpu.make_async_copy(
                    scratch.at[sl], o_hbm.at[b, sl], sems.at[h, b])
                c.start()
                copies.append(c)
        for c in copies:
            c.wait()

    out = pl.pallas_call(
        body,
        in_specs=[
            pl.BlockSpec(row_embed.shape, lambda: (0, 0)),
            pl.BlockSpec(col_embed.shape, lambda: (0, 0)),
        ],
        out_specs=pl.BlockSpec(memory_space=pl.ANY),
        out_shape=jax.ShapeDtypeStruct((B, H, W, 2 * D), jnp.float32),
        scratch_shapes=[
            pltpu.VMEM((H, W, 2 * D), jnp.float32),
            pltpu.SemaphoreType.DMA((NS, B)),
        ],
    )(row_embed, col_embed)
    return jnp.transpose(out, (0, 3, 1, 2))


def kernel(x, row_embed, col_embed):
    B = x.shape[0]
    H, W = x.shape[-2], x.shape[-1]
    return _pos_embed_tc(B, H, W, row_embed, col_embed)
